# Initial kernel scaffold; baseline (speedup 1.0000x reference)
import jax
import jax.numpy as jnp
from jax.experimental import pallas as pl


def kernel(feat, edge_index, node_type, edge_type, fc_W, fc_e_W, edge_emb, attn_l, attn_r, attn_e):
    raise NotImplementedError("write your pallas kernel here")


# speedup vs baseline: 29.9826x; 29.9826x over previous
import functools

import jax
import jax.numpy as jnp
from jax import lax
from jax.experimental import pallas as pl
from jax.experimental.pallas import tpu as pltpu
from jax.experimental.pallas import tpu_sc as plsc

N = 10000
E = 320000
D_IN = 128
H = 8
D_OUT = 16
D_E = 16
N_NT = 3
N_ET = 4
HD = H * D_OUT

NC = 2
NS = 16
NW = NC * NS
EPW = E // NW
C = 80
NCHUNK = EPW // C
N_PAD = 10240
RPS = N_PAD // NS

BR = 400
NB = N // BR


def _tc_proj_body(feat_ref, fcw_ref, wlr_ref, nt_ref, fs_ref, elr_ref):
    x = feat_ref[...]
    fs = jnp.dot(x, fcw_ref[...], preferred_element_type=jnp.float32)
    fs_ref[...] = fs
    nt = nt_ref[...]
    acc = jnp.zeros((BR, 2 * H), jnp.float32)
    for t in range(N_NT):
        elr_t = jnp.dot(fs, wlr_ref[t], preferred_element_type=jnp.float32)
        acc = acc + jnp.where(nt == t, elr_t, 0.0)
    elr_ref[...] = acc


def _tc_ee_body(ef_ref, fcew_ref, ae_ref, g_ref, out_ref):
    t1 = jnp.dot(ef_ref[...], fcew_ref[...], preferred_element_type=jnp.float32)
    out_ref[...] = jnp.dot(t1 * ae_ref[...], g_ref[...],
                           preferred_element_type=jnp.float32)


def _tc_add_body(a_ref, b_ref, o_ref):
    o_ref[...] = a_ref[...] + b_ref[...]


def _hb(h):
    return jnp.full((16,), h, jnp.int32)


def _sc_a_body(src_h, dst_h, et_h, elr_h, ee_h, z8_h,
               ex_o, dp_o,
               src_v, dst_v, et_v, el_v, er_v, ee_v, ex_v, dsh, sem1, sem2):
    c = lax.axis_index("c")
    s = lax.axis_index("s")
    pltpu.sync_copy(z8_h.at[pl.ds(s * RPS, RPS)], dsh.at[pl.ds(s * RPS, RPS)])
    pltpu.sync_copy(ee_h, ee_v)
    plsc.subcore_barrier()
    lane = lax.iota(jnp.int32, 16)
    base0 = (s * NC + c) * EPW

    def chunk(k, carry):
        base = base0 + k * C
        pltpu.sync_copy(src_h.at[pl.ds(base, C)], src_v)
        pltpu.sync_copy(dst_h.at[pl.ds(base, C)], dst_v)
        pltpu.sync_copy(et_h.at[pl.ds(base, C)], et_v)
        cp1 = pltpu.async_copy(elr_h.at[src_v], el_v, sem1)
        cp2 = pltpu.async_copy(elr_h.at[dst_v], er_v, sem2)
        cp1.wait()
        cp2.wait()

        def grp(j, carry2):
            rows = j * 16 + lane
            etv = et_v[pl.ds(j * 16, 16)]
            for h in range(H):
                elv = plsc.load_gather(el_v, [rows, _hb(h)])
                erv = plsc.load_gather(er_v, [rows, _hb(h + 8)])
                eev = plsc.load_gather(ee_v, [etv, _hb(h)])
                e = elv + erv + eev
                e = jnp.where(e >= 0.0, e, 0.2 * e)
                plsc.store_scatter(ex_v, [rows, _hb(h)], jnp.exp(e))
            return carry2

        lax.fori_loop(0, C // 16, grp, 0)
        pltpu.sync_copy(ex_v, ex_o.at[pl.ds(base, C)])
        pltpu.sync_copy(ex_v, dsh.at[dst_v], add=True)
        return carry

    lax.fori_loop(0, NCHUNK, chunk, 0)
    plsc.subcore_barrier()
    pltpu.sync_copy(dsh.at[pl.ds(s * RPS, RPS)],
                    dp_o.at[c, pl.ds(s * RPS, RPS)])


def _sc_b_body(src_h, dst_h, ex_h, d0_h, d1_h, fs_h, z128_h,
               a_o, rp_o,
               src_v, dst_v, ex_v, d0_v, d1_v, a_v, feat_v, rsh,
               sem1, sem2, sem3):
    c = lax.axis_index("c")
    s = lax.axis_index("s")
    pltpu.sync_copy(z128_h.at[pl.ds(s * RPS, RPS)], rsh.at[pl.ds(s * RPS, RPS)])
    plsc.subcore_barrier()
    lane = lax.iota(jnp.int32, 16)
    base0 = (s * NC + c) * EPW

    def chunk(k, carry):
        base = base0 + k * C
        pltpu.sync_copy(src_h.at[pl.ds(base, C)], src_v)
        pltpu.sync_copy(dst_h.at[pl.ds(base, C)], dst_v)
        g1 = pltpu.async_copy(fs_h.at[src_v], feat_v, sem1)
        pltpu.sync_copy(ex_h.at[pl.ds(base, C)], ex_v)
        g2 = pltpu.async_copy(d0_h.at[dst_v], d0_v, sem2)
        g3 = pltpu.async_copy(d1_h.at[dst_v], d1_v, sem3)
        g2.wait()
        g3.wait()

        def grp(j, carry2):
            rows = j * 16 + lane
            for h in range(H):
                exv = plsc.load_gather(ex_v, [rows, _hb(h)])
                dv = (plsc.load_gather(d0_v, [rows, _hb(h)])
                      + plsc.load_gather(d1_v, [rows, _hb(h)]))
                av = exv / jnp.maximum(dv, 1e-16)
                plsc.store_scatter(a_v, [rows, _hb(h)], av)
            return carry2

        lax.fori_loop(0, C // 16, grp, 0)
        pltpu.sync_copy(a_v, a_o.at[pl.ds(base, C)])
        g1.wait()

        def edge(i, carry2):
            eb = i + jnp.zeros((16,), jnp.int32)
            for h in range(H):
                asp = plsc.load_gather(a_v, [eb, jnp.full((16,), h, jnp.int32)])
                feat_v[i, pl.ds(h * D_OUT, D_OUT)] = (
                    feat_v[i, pl.ds(h * D_OUT, D_OUT)] * asp)
            return carry2

        lax.fori_loop(0, C, edge, 0)
        pltpu.sync_copy(feat_v, rsh.at[dst_v], add=True)
        return carry

    lax.fori_loop(0, NCHUNK, chunk, 0)
    plsc.subcore_barrier()
    pltpu.sync_copy(rsh.at[pl.ds(s * RPS, RPS)],
                    rp_o.at[c, pl.ds(s * RPS, RPS)])


def kernel(feat, edge_index, node_type, edge_type, fc_W, fc_e_W, edge_emb,
           attn_l, attn_r, attn_e):
    f32 = jnp.float32
    src = edge_index[0]
    dst = edge_index[1]
    nt_col = node_type.reshape(N, 1)

    eyeH = jnp.eye(H, dtype=f32)
    wl3 = jnp.einsum("thd,hk->thdk", attn_l, eyeH).reshape(N_NT, HD, H)
    wr3 = jnp.einsum("thd,hk->thdk", attn_r, eyeH).reshape(N_NT, HD, H)
    wlr = jnp.concatenate([wl3, wr3], axis=2)

    fs, elr = pl.pallas_call(
        _tc_proj_body,
        grid=(NB,),
        in_specs=[
            pl.BlockSpec((BR, D_IN), lambda i: (i, 0)),
            pl.BlockSpec((D_IN, HD), lambda i: (0, 0)),
            pl.BlockSpec((N_NT, HD, 2 * H), lambda i: (0, 0, 0)),
            pl.BlockSpec((BR, 1), lambda i: (i, 0)),
        ],
        out_specs=[
            pl.BlockSpec((BR, HD), lambda i: (i, 0)),
            pl.BlockSpec((BR, 2 * H), lambda i: (i, 0)),
        ],
        out_shape=[
            jax.ShapeDtypeStruct((N, HD), f32),
            jax.ShapeDtypeStruct((N, 2 * H), f32),
        ],
    )(feat, fc_W, wlr, nt_col)

    tf = jnp.arange(N_ET, dtype=f32)[:, None]
    ef_p = jnp.pad(tf * edge_emb, ((0, 8 - N_ET), (0, 0)))
    ae_p = jnp.pad(attn_e.reshape(N_ET, H * D_E), ((0, 8 - N_ET), (0, 0)))
    g_p = jnp.pad(jnp.repeat(eyeH, D_E, axis=0), ((0, 0), (0, HD - H)))
    ee_full = pl.pallas_call(
        _tc_ee_body,
        out_shape=jax.ShapeDtypeStruct((8, HD), f32),
    )(ef_p, fc_e_W, ae_p, g_p)
    ee_table = ee_full[:N_ET, :H]

    z8 = jnp.zeros((N_PAD, H), f32)
    z128 = jnp.zeros((N_PAD, HD), f32)
    mesh = plsc.VectorSubcoreMesh(core_axis_name="c", subcore_axis_name="s")
    sc_params = pltpu.CompilerParams(needs_layout_passes=False,
                                     use_tc_tiling_on_sc=False)

    pass_a = pl.kernel(
        _sc_a_body,
        mesh=mesh,
        compiler_params=sc_params,
        out_type=[
            jax.ShapeDtypeStruct((E, H), f32),
            jax.ShapeDtypeStruct((NC, N_PAD, H), f32),
        ],
        scratch_types=[
            pltpu.VMEM((C,), jnp.int32),
            pltpu.VMEM((C,), jnp.int32),
            pltpu.VMEM((C,), jnp.int32),
            pltpu.VMEM((C, 2 * H), f32),
            pltpu.VMEM((C, 2 * H), f32),
            pltpu.VMEM((N_ET, H), f32),
            pltpu.VMEM((C, H), f32),
            pltpu.VMEM_SHARED((N_PAD, H), f32),
            pltpu.SemaphoreType.DMA,
            pltpu.SemaphoreType.DMA,
        ],
    )
    ex, dparts = pass_a(src, dst, edge_type, elr, ee_table, z8)

    pass_b = pl.kernel(
        _sc_b_body,
        mesh=mesh,
        compiler_params=sc_params,
        out_type=[
            jax.ShapeDtypeStruct((E, H), f32),
            jax.ShapeDtypeStruct((NC, N_PAD, HD), f32),
        ],
        scratch_types=[
            pltpu.VMEM((C,), jnp.int32),
            pltpu.VMEM((C,), jnp.int32),
            pltpu.VMEM((C, H), f32),
            pltpu.VMEM((C, H), f32),
            pltpu.VMEM((C, H), f32),
            pltpu.VMEM((C, H), f32),
            pltpu.VMEM((C, HD), f32),
            pltpu.VMEM_SHARED((N_PAD, HD), f32),
            pltpu.SemaphoreType.DMA,
            pltpu.SemaphoreType.DMA,
            pltpu.SemaphoreType.DMA,
        ],
    )
    a_flat, rparts = pass_b(src, dst, ex, dparts[0], dparts[1], fs, z128)

    rst_flat = pl.pallas_call(
        _tc_add_body,
        grid=(NB,),
        in_specs=[pl.BlockSpec((BR, HD), lambda i: (i, 0)),
                  pl.BlockSpec((BR, HD), lambda i: (i, 0))],
        out_specs=pl.BlockSpec((BR, HD), lambda i: (i, 0)),
        out_shape=jax.ShapeDtypeStruct((N, HD), f32),
    )(rparts[0, :N], rparts[1, :N])

    return rst_flat.reshape(N, H, D_OUT), a_flat.reshape(E, H, 1)


# speedup vs baseline: 75.5625x; 2.5202x over previous
import functools

import jax
import jax.numpy as jnp
from jax import lax
from jax.experimental import pallas as pl
from jax.experimental.pallas import tpu as pltpu
from jax.experimental.pallas import tpu_sc as plsc

N = 10000
E = 320000
D_IN = 128
H = 8
D_OUT = 16
D_E = 16
N_NT = 3
N_ET = 4
HD = H * D_OUT

NC = 2
NS = 16
NW = NC * NS
EPW = E // NW
C = 80
NCHUNK = EPW // C
N_PAD = 10240
RPS = N_PAD // NS

BR = 400
NB = N // BR


def _tc_proj_body(feat_ref, fcw_ref, wlr_ref, nt_ref, fs_ref, elr_ref):
    x = feat_ref[...]
    fs = jnp.dot(x, fcw_ref[...], preferred_element_type=jnp.float32)
    fs_ref[...] = fs
    nt = nt_ref[...]
    acc = jnp.zeros((BR, 2 * H), jnp.float32)
    for t in range(N_NT):
        elr_t = jnp.dot(fs, wlr_ref[t], preferred_element_type=jnp.float32)
        acc = acc + jnp.where(nt == t, elr_t, 0.0)
    elr_ref[...] = acc


def _tc_ee_body(ef_ref, fcew_ref, ae_ref, g_ref, out_ref):
    t1 = jnp.dot(ef_ref[...], fcew_ref[...], preferred_element_type=jnp.float32)
    out_ref[...] = jnp.dot(t1 * ae_ref[...], g_ref[...],
                           preferred_element_type=jnp.float32)


def _tc_inv_body(dp_ref, inv_ref):
    inv_ref[...] = 1.0 / jnp.maximum(dp_ref[0] + dp_ref[1], 1e-16)


def _tc_add_body(a_ref, b_ref, o_ref):
    o_ref[...] = a_ref[...] + b_ref[...]


def _hb(h):
    return jnp.full((16,), h, jnp.int32)


def _sc_a_body(src3_h, dst3_h, et3_h, elr_h, ee_h, z8_h,
               ex_o, dp_o,
               six, dix, tix, el0, er0, el1, er1, ee_v, ex_v, dsh,
               sem0, sem1):
    c = lax.axis_index("c")
    s = lax.axis_index("s")
    wid = s * NC + c
    pltpu.sync_copy(z8_h.at[pl.ds(s * RPS, RPS)], dsh.at[pl.ds(s * RPS, RPS)])
    pltpu.sync_copy(ee_h, ee_v)
    pltpu.sync_copy(src3_h.at[wid], six)
    pltpu.sync_copy(dst3_h.at[wid], dix)
    pltpu.sync_copy(et3_h.at[wid], tix)
    plsc.subcore_barrier()
    lane = lax.iota(jnp.int32, 16)
    base0 = wid * EPW

    def issue(k, elb, erb, sem):
        pltpu.async_copy(elr_h.at[six.at[k]], elb, sem)
        pltpu.async_copy(elr_h.at[dix.at[k]], erb, sem)

    def wait(k, elb, erb, sem):
        pltpu.make_async_copy(elr_h.at[six.at[k]], elb, sem).wait()
        pltpu.make_async_copy(elr_h.at[dix.at[k]], erb, sem).wait()

    def compute(k, elb, erb):
        base = base0 + k * C

        def grp(j, carry2):
            rows = j * 16 + lane
            etv = tix[k, pl.ds(j * 16, 16)]
            for h in range(H):
                ev = (plsc.load_gather(elb, [rows, _hb(h)])
                      + plsc.load_gather(erb, [rows, _hb(h + 8)])
                      + plsc.load_gather(ee_v, [etv, _hb(h)]))
                ev = jnp.where(ev >= 0.0, ev, 0.2 * ev)
                plsc.store_scatter(ex_v, [rows, _hb(h)], jnp.exp(ev))
            return carry2

        lax.fori_loop(0, C // 16, grp, 0)
        pltpu.sync_copy(ex_v, ex_o.at[pl.ds(base, C)])
        pltpu.sync_copy(ex_v, dsh.at[dix.at[k]], add=True)

    issue(0, el0, er0, sem0)

    def body(i, carry):
        k0 = 2 * i
        issue(k0 + 1, el1, er1, sem1)
        wait(k0, el0, er0, sem0)
        compute(k0, el0, er0)
        issue(k0 + 2, el0, er0, sem0)
        wait(k0 + 1, el1, er1, sem1)
        compute(k0 + 1, el1, er1)
        return carry

    lax.fori_loop(0, (NCHUNK - 1) // 2, body, 0)
    wait(NCHUNK - 1, el0, er0, sem0)
    compute(NCHUNK - 1, el0, er0)
    plsc.subcore_barrier()
    pltpu.sync_copy(dsh.at[pl.ds(s * RPS, RPS)],
                    dp_o.at[c, pl.ds(s * RPS, RPS)])


def _sc_b_body(src3_h, dst3_h, ex_h, inv_h, fs_h, z128_h,
               a_o, rp_o,
               six, dix, ex0, iv0, f0, ex1, iv1, f1, a_v, rsh,
               sem0, sem1):
    c = lax.axis_index("c")
    s = lax.axis_index("s")
    wid = s * NC + c
    pltpu.sync_copy(z128_h.at[pl.ds(s * RPS, RPS)], rsh.at[pl.ds(s * RPS, RPS)])
    pltpu.sync_copy(src3_h.at[wid], six)
    pltpu.sync_copy(dst3_h.at[wid], dix)
    plsc.subcore_barrier()
    lane = lax.iota(jnp.int32, 16)
    base0 = wid * EPW

    def issue(k, fb, eb, ib, sem):
        base = base0 + k * C
        pltpu.async_copy(fs_h.at[six.at[k]], fb, sem)
        pltpu.async_copy(ex_h.at[pl.ds(base, C)], eb, sem)
        pltpu.async_copy(inv_h.at[dix.at[k]], ib, sem)

    def wait(k, fb, eb, ib, sem):
        base = base0 + k * C
        pltpu.make_async_copy(fs_h.at[six.at[k]], fb, sem).wait()
        pltpu.make_async_copy(ex_h.at[pl.ds(base, C)], eb, sem).wait()
        pltpu.make_async_copy(inv_h.at[dix.at[k]], ib, sem).wait()

    def compute(k, fb, eb, ib):
        base = base0 + k * C

        def grp(j, carry2):
            rows = j * 16 + lane
            r2 = rows >> 1
            cb = (rows & 1) * 8
            for h in range(H):
                av = (plsc.load_gather(eb, [rows, _hb(h)])
                      * plsc.load_gather(ib, [rows, _hb(h)]))
                plsc.store_scatter(a_v, [r2, cb + _hb(h)], av)
            return carry2

        lax.fori_loop(0, C // 16, grp, 0)
        pltpu.sync_copy(a_v, a_o.at[pl.ds(base // 2, C // 2)])

        def pair(j, carry2):
            arow = a_v[j, :]
            for e01 in range(2):
                e = 2 * j + e01
                for h in range(H):
                    asp = jnp.broadcast_to(arow[e01 * 8 + h], (16,))
                    fb[e, pl.ds(h * D_OUT, D_OUT)] = (
                        fb[e, pl.ds(h * D_OUT, D_OUT)] * asp)
            return carry2

        lax.fori_loop(0, C // 2, pair, 0)
        pltpu.sync_copy(fb, rsh.at[dix.at[k]], add=True)

    issue(0, f0, ex0, iv0, sem0)

    def body(i, carry):
        k0 = 2 * i
        issue(k0 + 1, f1, ex1, iv1, sem1)
        wait(k0, f0, ex0, iv0, sem0)
        compute(k0, f0, ex0, iv0)
        issue(k0 + 2, f0, ex0, iv0, sem0)
        wait(k0 + 1, f1, ex1, iv1, sem1)
        compute(k0 + 1, f1, ex1, iv1)
        return carry

    lax.fori_loop(0, (NCHUNK - 1) // 2, body, 0)
    wait(NCHUNK - 1, f0, ex0, iv0, sem0)
    compute(NCHUNK - 1, f0, ex0, iv0)
    plsc.subcore_barrier()
    pltpu.sync_copy(rsh.at[pl.ds(s * RPS, RPS)],
                    rp_o.at[c, pl.ds(s * RPS, RPS)])


def kernel(feat, edge_index, node_type, edge_type, fc_W, fc_e_W, edge_emb,
           attn_l, attn_r, attn_e):
    f32 = jnp.float32
    src3 = edge_index[0].reshape(NW, NCHUNK, C)
    dst3 = edge_index[1].reshape(NW, NCHUNK, C)
    et3 = edge_type.reshape(NW, NCHUNK, C)
    nt_col = node_type.reshape(N, 1)

    eyeH = jnp.eye(H, dtype=f32)
    wl3 = jnp.einsum("thd,hk->thdk", attn_l, eyeH).reshape(N_NT, HD, H)
    wr3 = jnp.einsum("thd,hk->thdk", attn_r, eyeH).reshape(N_NT, HD, H)
    wlr = jnp.concatenate([wl3, wr3], axis=2)

    fs, elr = pl.pallas_call(
        _tc_proj_body,
        grid=(NB,),
        in_specs=[
            pl.BlockSpec((BR, D_IN), lambda i: (i, 0)),
            pl.BlockSpec((D_IN, HD), lambda i: (0, 0)),
            pl.BlockSpec((N_NT, HD, 2 * H), lambda i: (0, 0, 0)),
            pl.BlockSpec((BR, 1), lambda i: (i, 0)),
        ],
        out_specs=[
            pl.BlockSpec((BR, HD), lambda i: (i, 0)),
            pl.BlockSpec((BR, 2 * H), lambda i: (i, 0)),
        ],
        out_shape=[
            jax.ShapeDtypeStruct((N, HD), f32),
            jax.ShapeDtypeStruct((N, 2 * H), f32),
        ],
    )(feat, fc_W, wlr, nt_col)

    tf = jnp.arange(N_ET, dtype=f32)[:, None]
    ef_p = jnp.pad(tf * edge_emb, ((0, 8 - N_ET), (0, 0)))
    ae_p = jnp.pad(attn_e.reshape(N_ET, H * D_E), ((0, 8 - N_ET), (0, 0)))
    g_p = jnp.pad(jnp.repeat(eyeH, D_E, axis=0), ((0, 0), (0, HD - H)))
    ee_full = pl.pallas_call(
        _tc_ee_body,
        out_shape=jax.ShapeDtypeStruct((8, HD), f32),
    )(ef_p, fc_e_W, ae_p, g_p)
    ee_table = ee_full[:N_ET, :H]

    z8 = jnp.zeros((N_PAD, H), f32)
    z128 = jnp.zeros((N_PAD, HD), f32)
    mesh = plsc.VectorSubcoreMesh(core_axis_name="c", subcore_axis_name="s")
    sc_params = pltpu.CompilerParams(needs_layout_passes=False,
                                     use_tc_tiling_on_sc=False)

    pass_a = pl.kernel(
        _sc_a_body,
        mesh=mesh,
        compiler_params=sc_params,
        out_type=[
            jax.ShapeDtypeStruct((E, H), f32),
            jax.ShapeDtypeStruct((NC, N_PAD, H), f32),
        ],
        scratch_types=[
            pltpu.VMEM((NCHUNK, C), jnp.int32),
            pltpu.VMEM((NCHUNK, C), jnp.int32),
            pltpu.VMEM((NCHUNK, C), jnp.int32),
            pltpu.VMEM((C, 2 * H), f32),
            pltpu.VMEM((C, 2 * H), f32),
            pltpu.VMEM((C, 2 * H), f32),
            pltpu.VMEM((C, 2 * H), f32),
            pltpu.VMEM((N_ET, H), f32),
            pltpu.VMEM((C, H), f32),
            pltpu.VMEM_SHARED((N_PAD, H), f32),
            pltpu.SemaphoreType.DMA,
            pltpu.SemaphoreType.DMA,
        ],
    )
    ex, dparts = pass_a(src3, dst3, et3, elr, ee_table, z8)

    inv = pl.pallas_call(
        _tc_inv_body,
        grid=(8,),
        in_specs=[pl.BlockSpec((NC, N_PAD // 8, H), lambda i: (0, i, 0))],
        out_specs=pl.BlockSpec((N_PAD // 8, H), lambda i: (i, 0)),
        out_shape=jax.ShapeDtypeStruct((N_PAD, H), f32),
    )(dparts)

    pass_b = pl.kernel(
        _sc_b_body,
        mesh=mesh,
        compiler_params=sc_params,
        out_type=[
            jax.ShapeDtypeStruct((E // 2, 2 * H), f32),
            jax.ShapeDtypeStruct((NC, N_PAD, HD), f32),
        ],
        scratch_types=[
            pltpu.VMEM((NCHUNK, C), jnp.int32),
            pltpu.VMEM((NCHUNK, C), jnp.int32),
            pltpu.VMEM((C, H), f32),
            pltpu.VMEM((C, H), f32),
            pltpu.VMEM((C, HD), f32),
            pltpu.VMEM((C, H), f32),
            pltpu.VMEM((C, H), f32),
            pltpu.VMEM((C, HD), f32),
            pltpu.VMEM((C // 2, 16), f32),
            pltpu.VMEM_SHARED((N_PAD, HD), f32),
            pltpu.SemaphoreType.DMA,
            pltpu.SemaphoreType.DMA,
        ],
    )
    a_flat, rparts = pass_b(src3, dst3, ex, inv, fs, z128)

    rst_flat = pl.pallas_call(
        _tc_add_body,
        grid=(NB,),
        in_specs=[pl.BlockSpec((BR, HD), lambda i: (i, 0)),
                  pl.BlockSpec((BR, HD), lambda i: (i, 0))],
        out_specs=pl.BlockSpec((BR, HD), lambda i: (i, 0)),
        out_shape=jax.ShapeDtypeStruct((N, HD), f32),
    )(rparts[0, :N], rparts[1, :N])

    return rst_flat.reshape(N, H, D_OUT), a_flat.reshape(E, H, 1)


# speedup vs baseline: 77.1400x; 1.0209x over previous
import functools

import jax
import jax.numpy as jnp
from jax import lax
from jax.experimental import pallas as pl
from jax.experimental.pallas import tpu as pltpu
from jax.experimental.pallas import tpu_sc as plsc

N = 10000
E = 320000
D_IN = 128
H = 8
D_OUT = 16
D_E = 16
N_NT = 3
N_ET = 4
HD = H * D_OUT

NC = 2
NS = 16
NW = NC * NS
EPW = E // NW
C = 80
NCHUNK = EPW // C
N_PAD = 10240
RPS = N_PAD // NS

BR = 400
NB = N // BR


def _tc_proj_body(feat_ref, fcw_ref, wlr_ref, nt_ref, ef_ref, fcew_ref,
                  ae_ref, g_ref, fs_ref, elr_ref, ee_ref):
    x = feat_ref[...]
    fs = jnp.dot(x, fcw_ref[...], preferred_element_type=jnp.float32)
    fs_ref[...] = fs
    nt = nt_ref[...]
    acc = jnp.zeros((BR, 2 * H), jnp.float32)
    for t in range(N_NT):
        elr_t = jnp.dot(fs, wlr_ref[t], preferred_element_type=jnp.float32)
        acc = acc + jnp.where(nt == t, elr_t, 0.0)
    elr_ref[...] = acc

    @pl.when(pl.program_id(0) == 0)
    def _():
        t1 = jnp.dot(ef_ref[...], fcew_ref[...],
                     preferred_element_type=jnp.float32)
        ee_ref[...] = jnp.dot(t1 * ae_ref[...], g_ref[...],
                              preferred_element_type=jnp.float32)


def _tc_inv_body(dp_ref, inv_ref):
    inv_ref[...] = 1.0 / jnp.maximum(dp_ref[0] + dp_ref[1], 1e-16)


def _tc_add_body(rp_ref, o_ref):
    o_ref[...] = rp_ref[0] + rp_ref[1]


def _hb(h):
    return jnp.full((16,), h, jnp.int32)


def _sc_a_body(src3_h, dst3_h, et3_h, elr_h, ee_h, z8_h,
               ex_o, dp_o,
               six, dix, tix, el0, er0, el1, er1, ee_v, ex_v, dsh,
               sem0, sem1):
    c = lax.axis_index("c")
    s = lax.axis_index("s")
    wid = s * NC + c
    pltpu.sync_copy(z8_h.at[pl.ds(s * RPS, RPS)], dsh.at[pl.ds(s * RPS, RPS)])
    pltpu.sync_copy(ee_h, ee_v)
    pltpu.sync_copy(src3_h.at[wid], six)
    pltpu.sync_copy(dst3_h.at[wid], dix)
    pltpu.sync_copy(et3_h.at[wid], tix)
    plsc.subcore_barrier()
    lane = lax.iota(jnp.int32, 16)
    base0 = wid * EPW

    def issue(k, elb, erb, sem):
        pltpu.async_copy(elr_h.at[six.at[k]], elb, sem)
        pltpu.async_copy(elr_h.at[dix.at[k]], erb, sem)

    def wait(k, elb, erb, sem):
        pltpu.make_async_copy(elr_h.at[six.at[k]], elb, sem).wait()
        pltpu.make_async_copy(elr_h.at[dix.at[k]], erb, sem).wait()

    def compute(k, elb, erb):
        base = base0 + k * C

        def grp(j, carry2):
            rows = j * 16 + lane
            etv = tix[k, pl.ds(j * 16, 16)]
            for h in range(H):
                ev = (plsc.load_gather(elb, [rows, _hb(h)])
                      + plsc.load_gather(erb, [rows, _hb(h + 8)])
                      + plsc.load_gather(ee_v, [etv, _hb(h)]))
                ev = jnp.where(ev >= 0.0, ev, 0.2 * ev)
                plsc.store_scatter(ex_v, [rows, _hb(h)], jnp.exp(ev))
            return carry2

        lax.fori_loop(0, C // 16, grp, 0)
        pltpu.sync_copy(ex_v, ex_o.at[pl.ds(base, C)])
        pltpu.sync_copy(ex_v, dsh.at[dix.at[k]], add=True)

    issue(0, el0, er0, sem0)

    def body(i, carry):
        k0 = 2 * i
        issue(k0 + 1, el1, er1, sem1)
        wait(k0, el0, er0, sem0)
        compute(k0, el0, er0)
        issue(k0 + 2, el0, er0, sem0)
        wait(k0 + 1, el1, er1, sem1)
        compute(k0 + 1, el1, er1)
        return carry

    lax.fori_loop(0, (NCHUNK - 1) // 2, body, 0)
    wait(NCHUNK - 1, el0, er0, sem0)
    compute(NCHUNK - 1, el0, er0)
    plsc.subcore_barrier()
    pltpu.sync_copy(dsh.at[pl.ds(s * RPS, RPS)],
                    dp_o.at[c, pl.ds(s * RPS, RPS)])


def _sc_b_body(src3_h, dst3_h, ex_h, inv_h, fs_h,
               a_o, rp_o,
               six, dix, ex0, iv0, f0, ex1, iv1, f1, a_v, rsh,
               sem0, sem1):
    c = lax.axis_index("c")
    s = lax.axis_index("s")
    wid = s * NC + c
    zv = jnp.zeros((16,), jnp.float32)

    def zrow(r, carry):
        for h in range(H):
            f0[r, pl.ds(h * D_OUT, D_OUT)] = zv
        return carry

    lax.fori_loop(0, C, zrow, 0)
    for m in range(RPS // C):
        pltpu.sync_copy(f0, rsh.at[pl.ds(s * RPS + m * C, C)])
    pltpu.sync_copy(src3_h.at[wid], six)
    pltpu.sync_copy(dst3_h.at[wid], dix)
    plsc.subcore_barrier()
    lane = lax.iota(jnp.int32, 16)
    base0 = wid * EPW

    def issue(k, fb, eb, ib, sem):
        base = base0 + k * C
        pltpu.async_copy(fs_h.at[six.at[k]], fb, sem)
        pltpu.async_copy(ex_h.at[pl.ds(base, C)], eb, sem)
        pltpu.async_copy(inv_h.at[dix.at[k]], ib, sem)

    def wait(k, fb, eb, ib, sem):
        base = base0 + k * C
        pltpu.make_async_copy(fs_h.at[six.at[k]], fb, sem).wait()
        pltpu.make_async_copy(ex_h.at[pl.ds(base, C)], eb, sem).wait()
        pltpu.make_async_copy(inv_h.at[dix.at[k]], ib, sem).wait()

    def compute(k, fb, eb, ib):
        base = base0 + k * C

        def grp(j, carry2):
            rows = j * 16 + lane
            r2 = rows >> 1
            cb = (rows & 1) * 8
            for h in range(H):
                av = (plsc.load_gather(eb, [rows, _hb(h)])
                      * plsc.load_gather(ib, [rows, _hb(h)]))
                plsc.store_scatter(a_v, [r2, cb + _hb(h)], av)
            return carry2

        lax.fori_loop(0, C // 16, grp, 0)
        pltpu.sync_copy(a_v, a_o.at[pl.ds(base // 2, C // 2)])

        def pair(j, carry2):
            arow = a_v[j, :]
            for e01 in range(2):
                e = 2 * j + e01
                for h in range(H):
                    asp = jnp.broadcast_to(arow[e01 * 8 + h], (16,))
                    fb[e, pl.ds(h * D_OUT, D_OUT)] = (
                        fb[e, pl.ds(h * D_OUT, D_OUT)] * asp)
            return carry2

        lax.fori_loop(0, C // 2, pair, 0)
        pltpu.sync_copy(fb, rsh.at[dix.at[k]], add=True)

    issue(0, f0, ex0, iv0, sem0)

    def body(i, carry):
        k0 = 2 * i
        issue(k0 + 1, f1, ex1, iv1, sem1)
        wait(k0, f0, ex0, iv0, sem0)
        compute(k0, f0, ex0, iv0)
        issue(k0 + 2, f0, ex0, iv0, sem0)
        wait(k0 + 1, f1, ex1, iv1, sem1)
        compute(k0 + 1, f1, ex1, iv1)
        return carry

    lax.fori_loop(0, (NCHUNK - 1) // 2, body, 0)
    wait(NCHUNK - 1, f0, ex0, iv0, sem0)
    compute(NCHUNK - 1, f0, ex0, iv0)
    plsc.subcore_barrier()
    pltpu.sync_copy(rsh.at[pl.ds(s * RPS, RPS)],
                    rp_o.at[c, pl.ds(s * RPS, RPS)])


def kernel(feat, edge_index, node_type, edge_type, fc_W, fc_e_W, edge_emb,
           attn_l, attn_r, attn_e):
    f32 = jnp.float32
    src3 = edge_index[0].reshape(NW, NCHUNK, C)
    dst3 = edge_index[1].reshape(NW, NCHUNK, C)
    et3 = edge_type.reshape(NW, NCHUNK, C)
    nt_col = node_type.reshape(N, 1)

    eyeH = jnp.eye(H, dtype=f32)
    wl3 = jnp.einsum("thd,hk->thdk", attn_l, eyeH).reshape(N_NT, HD, H)
    wr3 = jnp.einsum("thd,hk->thdk", attn_r, eyeH).reshape(N_NT, HD, H)
    wlr = jnp.concatenate([wl3, wr3], axis=2)

    tf = jnp.arange(N_ET, dtype=f32)[:, None]
    ef_p = jnp.pad(tf * edge_emb, ((0, 8 - N_ET), (0, 0)))
    ae_p = jnp.pad(attn_e.reshape(N_ET, H * D_E), ((0, 8 - N_ET), (0, 0)))
    g_p = jnp.pad(jnp.repeat(eyeH, D_E, axis=0), ((0, 0), (0, HD - H)))

    fs, elr, ee_full = pl.pallas_call(
        _tc_proj_body,
        grid=(NB,),
        in_specs=[
            pl.BlockSpec((BR, D_IN), lambda i: (i, 0)),
            pl.BlockSpec((D_IN, HD), lambda i: (0, 0)),
            pl.BlockSpec((N_NT, HD, 2 * H), lambda i: (0, 0, 0)),
            pl.BlockSpec((BR, 1), lambda i: (i, 0)),
            pl.BlockSpec((8, D_E), lambda i: (0, 0)),
            pl.BlockSpec((D_E, H * D_E), lambda i: (0, 0)),
            pl.BlockSpec((8, H * D_E), lambda i: (0, 0)),
            pl.BlockSpec((H * D_E, HD), lambda i: (0, 0)),
        ],
        out_specs=[
            pl.BlockSpec((BR, HD), lambda i: (i, 0)),
            pl.BlockSpec((BR, 2 * H), lambda i: (i, 0)),
            pl.BlockSpec((8, HD), lambda i: (0, 0)),
        ],
        out_shape=[
            jax.ShapeDtypeStruct((N, HD), f32),
            jax.ShapeDtypeStruct((N, 2 * H), f32),
            jax.ShapeDtypeStruct((8, HD), f32),
        ],
    )(feat, fc_W, wlr, nt_col, ef_p, fc_e_W, ae_p, g_p)
    ee_table = ee_full[:N_ET, :H]

    z8 = jnp.zeros((N_PAD, H), f32)
    mesh = plsc.VectorSubcoreMesh(core_axis_name="c", subcore_axis_name="s")
    sc_params = pltpu.CompilerParams(needs_layout_passes=False,
                                     use_tc_tiling_on_sc=False)

    pass_a = pl.kernel(
        _sc_a_body,
        mesh=mesh,
        compiler_params=sc_params,
        out_type=[
            jax.ShapeDtypeStruct((E, H), f32),
            jax.ShapeDtypeStruct((NC, N_PAD, H), f32),
        ],
        scratch_types=[
            pltpu.VMEM((NCHUNK, C), jnp.int32),
            pltpu.VMEM((NCHUNK, C), jnp.int32),
            pltpu.VMEM((NCHUNK, C), jnp.int32),
            pltpu.VMEM((C, 2 * H), f32),
            pltpu.VMEM((C, 2 * H), f32),
            pltpu.VMEM((C, 2 * H), f32),
            pltpu.VMEM((C, 2 * H), f32),
            pltpu.VMEM((N_ET, H), f32),
            pltpu.VMEM((C, H), f32),
            pltpu.VMEM_SHARED((N_PAD, H), f32),
            pltpu.SemaphoreType.DMA,
            pltpu.SemaphoreType.DMA,
        ],
    )
    ex, dparts = pass_a(src3, dst3, et3, elr, ee_table, z8)

    inv = pl.pallas_call(
        _tc_inv_body,
        grid=(8,),
        in_specs=[pl.BlockSpec((NC, N_PAD // 8, H), lambda i: (0, i, 0))],
        out_specs=pl.BlockSpec((N_PAD // 8, H), lambda i: (i, 0)),
        out_shape=jax.ShapeDtypeStruct((N_PAD, H), f32),
    )(dparts)

    pass_b = pl.kernel(
        _sc_b_body,
        mesh=mesh,
        compiler_params=sc_params,
        out_type=[
            jax.ShapeDtypeStruct((E // 2, 2 * H), f32),
            jax.ShapeDtypeStruct((NC, N_PAD, HD), f32),
        ],
        scratch_types=[
            pltpu.VMEM((NCHUNK, C), jnp.int32),
            pltpu.VMEM((NCHUNK, C), jnp.int32),
            pltpu.VMEM((C, H), f32),
            pltpu.VMEM((C, H), f32),
            pltpu.VMEM((C, HD), f32),
            pltpu.VMEM((C, H), f32),
            pltpu.VMEM((C, H), f32),
            pltpu.VMEM((C, HD), f32),
            pltpu.VMEM((C // 2, 16), f32),
            pltpu.VMEM_SHARED((N_PAD, HD), f32),
            pltpu.SemaphoreType.DMA,
            pltpu.SemaphoreType.DMA,
        ],
    )
    a_flat, rparts = pass_b(src3, dst3, ex, inv, fs)

    rst_flat = pl.pallas_call(
        _tc_add_body,
        grid=(NB,),
        in_specs=[pl.BlockSpec((NC, BR, HD), lambda i: (0, i, 0))],
        out_specs=pl.BlockSpec((BR, HD), lambda i: (i, 0)),
        out_shape=jax.ShapeDtypeStruct((N, HD), f32),
    )(rparts)

    return rst_flat.reshape(N, H, D_OUT), a_flat.reshape(E, H, 1)


# speedup vs baseline: 79.6059x; 1.0320x over previous
import functools

import jax
import jax.numpy as jnp
from jax import lax
from jax.experimental import pallas as pl
from jax.experimental.pallas import tpu as pltpu
from jax.experimental.pallas import tpu_sc as plsc

N = 10000
E = 320000
D_IN = 128
H = 8
D_OUT = 16
D_E = 16
N_NT = 3
N_ET = 4
HD = H * D_OUT

NC = 2
NS = 16
NW = NC * NS
EPW = E // NW
C = 80
NCHUNK = EPW // C
N_PAD = 10240
RPS = N_PAD // NS

BR = 400
NB = N // BR


def _tc_proj_body(feat_ref, fcw_ref, wlr_ref, nt_ref, ef_ref, fcew_ref,
                  ae_ref, g_ref, fs_ref, elr_ref, ee_ref):
    x = feat_ref[...]
    fs = jnp.dot(x, fcw_ref[...], preferred_element_type=jnp.float32)
    fs_ref[...] = fs
    nt = nt_ref[...]
    acc = jnp.zeros((BR, 2 * H), jnp.float32)
    for t in range(N_NT):
        elr_t = jnp.dot(fs, wlr_ref[t], preferred_element_type=jnp.float32)
        acc = acc + jnp.where(nt == t, elr_t, 0.0)
    elr_ref[...] = acc

    @pl.when(pl.program_id(0) == 0)
    def _():
        t1 = jnp.dot(ef_ref[...], fcew_ref[...],
                     preferred_element_type=jnp.float32)
        ee_ref[...] = jnp.dot(t1 * ae_ref[...], g_ref[...],
                              preferred_element_type=jnp.float32)


def _tc_inv_body(dp_ref, inv_ref):
    inv_ref[...] = 1.0 / jnp.maximum(dp_ref[0] + dp_ref[1], 1e-16)


def _tc_add_body(rp_ref, o_ref):
    o_ref[...] = rp_ref[0] + rp_ref[1]


def _hb(h):
    return jnp.full((16,), h, jnp.int32)


def _sc_a_body(src3_h, dst3_h, et3_h, elr_h, ee_h, z8_h,
               ex_o, dp_o,
               six, dix, tix, el0, er0, el1, er1, ee_v, ex_v, dsh,
               sem0, sem1, sem2):
    c = lax.axis_index("c")
    s = lax.axis_index("s")
    wid = s * NC + c
    lane = lax.iota(jnp.int32, 16)
    pltpu.sync_copy(z8_h.at[pl.ds(s * RPS, RPS)], dsh.at[pl.ds(s * RPS, RPS)])
    pltpu.sync_copy(ee_h, ee_v)
    pltpu.sync_copy(src3_h.at[wid], six)
    pltpu.sync_copy(dst3_h.at[wid], dix)
    pltpu.sync_copy(et3_h.at[wid], tix)
    plsc.subcore_barrier()
    base0 = wid * EPW

    def issue(k, elb, erb, sem):
        pltpu.async_copy(elr_h.at[six.at[k]], elb, sem)
        pltpu.async_copy(elr_h.at[dix.at[k]], erb, sem)

    def wait(k, elb, erb, sem):
        pltpu.make_async_copy(elr_h.at[six.at[k]], elb, sem).wait()
        pltpu.make_async_copy(elr_h.at[dix.at[k]], erb, sem).wait()

    def compute(k, elb, erb):
        base = base0 + k * C

        def grp(j, carry2):
            rows = j * 16 + lane
            etv = tix[k, pl.ds(j * 16, 16)]
            for h in range(H):
                ev = (plsc.load_gather(elb, [rows, _hb(h)])
                      + plsc.load_gather(erb, [rows, _hb(h + 8)])
                      + plsc.load_gather(ee_v, [etv, _hb(h)]))
                ev = jnp.where(ev >= 0.0, ev, 0.2 * ev)
                plsc.store_scatter(ex_v, [rows, _hb(h)], jnp.exp(ev))
            return carry2

        lax.fori_loop(0, C // 16, grp, 0)
        pltpu.async_copy(ex_v, ex_o.at[pl.ds(base, C)], sem2)
        pltpu.sync_copy(ex_v, dsh.at[dix.at[k]], add=True)
        pltpu.make_async_copy(ex_v, ex_o.at[pl.ds(base, C)], sem2).wait()

    issue(0, el0, er0, sem0)

    def body(i, carry):
        k0 = 2 * i
        issue(k0 + 1, el1, er1, sem1)
        wait(k0, el0, er0, sem0)
        compute(k0, el0, er0)
        issue(k0 + 2, el0, er0, sem0)
        wait(k0 + 1, el1, er1, sem1)
        compute(k0 + 1, el1, er1)
        return carry

    lax.fori_loop(0, (NCHUNK - 1) // 2, body, 0)
    wait(NCHUNK - 1, el0, er0, sem0)
    compute(NCHUNK - 1, el0, er0)
    plsc.subcore_barrier()
    pltpu.sync_copy(dsh.at[pl.ds(s * RPS, RPS)],
                    dp_o.at[c, pl.ds(s * RPS, RPS)])


def _sc_b_body(src3_h, dst3_h, ex_h, inv_h, fs_h,
               a_o, rp_o,
               six, dix, ex0, iv0, f0, ex1, iv1, f1, a_v, rsh,
               sem0, sem1, sem2):
    c = lax.axis_index("c")
    s = lax.axis_index("s")
    wid = s * NC + c
    zv = jnp.zeros((16,), jnp.float32)

    def zrow(r, carry):
        for h in range(H):
            f0[r, pl.ds(h * D_OUT, D_OUT)] = zv
        return carry

    lax.fori_loop(0, C, zrow, 0)
    for m in range(RPS // C):
        pltpu.sync_copy(f0, rsh.at[pl.ds(s * RPS + m * C, C)])
    pltpu.sync_copy(src3_h.at[wid], six)
    pltpu.sync_copy(dst3_h.at[wid], dix)
    plsc.subcore_barrier()
    lane = lax.iota(jnp.int32, 16)
    base0 = wid * EPW

    def issue(k, fb, eb, ib, sem):
        base = base0 + k * C
        pltpu.async_copy(fs_h.at[six.at[k]], fb, sem)
        pltpu.async_copy(ex_h.at[pl.ds(base, C)], eb, sem)
        pltpu.async_copy(inv_h.at[dix.at[k]], ib, sem)

    def wait(k, fb, eb, ib, sem):
        base = base0 + k * C
        pltpu.make_async_copy(fs_h.at[six.at[k]], fb, sem).wait()
        pltpu.make_async_copy(ex_h.at[pl.ds(base, C)], eb, sem).wait()
        pltpu.make_async_copy(inv_h.at[dix.at[k]], ib, sem).wait()

    def compute(k, fb, eb, ib):
        base = base0 + k * C

        def grp(j, carry2):
            rows = j * 16 + lane
            r2 = rows >> 1
            cb = (rows & 1) * 8
            for h in range(H):
                av = (plsc.load_gather(eb, [rows, _hb(h)])
                      * plsc.load_gather(ib, [rows, _hb(h)]))
                plsc.store_scatter(a_v, [r2, cb + _hb(h)], av)
            return carry2

        lax.fori_loop(0, C // 16, grp, 0)
        pltpu.async_copy(a_v, a_o.at[pl.ds(base // 2, C // 2)], sem2)

        def pair(j, carry2):
            arow = a_v[j, :]
            for e01 in range(2):
                e = 2 * j + e01
                for h in range(H):
                    asp = jnp.broadcast_to(arow[e01 * 8 + h], (16,))
                    fb[e, pl.ds(h * D_OUT, D_OUT)] = (
                        fb[e, pl.ds(h * D_OUT, D_OUT)] * asp)
            return carry2

        lax.fori_loop(0, C // 2, pair, 0)
        pltpu.sync_copy(fb, rsh.at[dix.at[k]], add=True)
        pltpu.make_async_copy(a_v, a_o.at[pl.ds(base // 2, C // 2)], sem2).wait()

    issue(0, f0, ex0, iv0, sem0)

    def body(i, carry):
        k0 = 2 * i
        issue(k0 + 1, f1, ex1, iv1, sem1)
        wait(k0, f0, ex0, iv0, sem0)
        compute(k0, f0, ex0, iv0)
        issue(k0 + 2, f0, ex0, iv0, sem0)
        wait(k0 + 1, f1, ex1, iv1, sem1)
        compute(k0 + 1, f1, ex1, iv1)
        return carry

    lax.fori_loop(0, (NCHUNK - 1) // 2, body, 0)
    wait(NCHUNK - 1, f0, ex0, iv0, sem0)
    compute(NCHUNK - 1, f0, ex0, iv0)
    plsc.subcore_barrier()
    pltpu.sync_copy(rsh.at[pl.ds(s * RPS, RPS)],
                    rp_o.at[c, pl.ds(s * RPS, RPS)])


def kernel(feat, edge_index, node_type, edge_type, fc_W, fc_e_W, edge_emb,
           attn_l, attn_r, attn_e):
    f32 = jnp.float32
    src3 = edge_index[0].reshape(NW, NCHUNK, C)
    dst3 = edge_index[1].reshape(NW, NCHUNK, C)
    et3 = edge_type.reshape(NW, NCHUNK, C)
    nt_col = node_type.reshape(N, 1)

    eyeH = jnp.eye(H, dtype=f32)
    wl3 = jnp.einsum("thd,hk->thdk", attn_l, eyeH).reshape(N_NT, HD, H)
    wr3 = jnp.einsum("thd,hk->thdk", attn_r, eyeH).reshape(N_NT, HD, H)
    wlr = jnp.concatenate([wl3, wr3], axis=2)

    tf = jnp.arange(N_ET, dtype=f32)[:, None]
    ef_p = jnp.pad(tf * edge_emb, ((0, 8 - N_ET), (0, 0)))
    ae_p = jnp.pad(attn_e.reshape(N_ET, H * D_E), ((0, 8 - N_ET), (0, 0)))
    g_p = jnp.pad(jnp.repeat(eyeH, D_E, axis=0), ((0, 0), (0, HD - H)))

    fs, elr, ee_full = pl.pallas_call(
        _tc_proj_body,
        grid=(NB,),
        in_specs=[
            pl.BlockSpec((BR, D_IN), lambda i: (i, 0)),
            pl.BlockSpec((D_IN, HD), lambda i: (0, 0)),
            pl.BlockSpec((N_NT, HD, 2 * H), lambda i: (0, 0, 0)),
            pl.BlockSpec((BR, 1), lambda i: (i, 0)),
            pl.BlockSpec((8, D_E), lambda i: (0, 0)),
            pl.BlockSpec((D_E, H * D_E), lambda i: (0, 0)),
            pl.BlockSpec((8, H * D_E), lambda i: (0, 0)),
            pl.BlockSpec((H * D_E, HD), lambda i: (0, 0)),
        ],
        out_specs=[
            pl.BlockSpec((BR, HD), lambda i: (i, 0)),
            pl.BlockSpec((BR, 2 * H), lambda i: (i, 0)),
            pl.BlockSpec((8, HD), lambda i: (0, 0)),
        ],
        out_shape=[
            jax.ShapeDtypeStruct((N, HD), f32),
            jax.ShapeDtypeStruct((N, 2 * H), f32),
            jax.ShapeDtypeStruct((8, HD), f32),
        ],
    )(feat, fc_W, wlr, nt_col, ef_p, fc_e_W, ae_p, g_p)
    ee_table = ee_full[:N_ET, :H]

    z8 = jnp.zeros((N_PAD, H), f32)
    mesh = plsc.VectorSubcoreMesh(core_axis_name="c", subcore_axis_name="s")
    sc_params = pltpu.CompilerParams(needs_layout_passes=False,
                                     use_tc_tiling_on_sc=False)

    pass_a = pl.kernel(
        _sc_a_body,
        mesh=mesh,
        compiler_params=sc_params,
        out_type=[
            jax.ShapeDtypeStruct((E, H), f32),
            jax.ShapeDtypeStruct((NC, N_PAD, H), f32),
        ],
        scratch_types=[
            pltpu.VMEM((NCHUNK, C), jnp.int32),
            pltpu.VMEM((NCHUNK, C), jnp.int32),
            pltpu.VMEM((NCHUNK, C), jnp.int32),
            pltpu.VMEM((C, 2 * H), f32),
            pltpu.VMEM((C, 2 * H), f32),
            pltpu.VMEM((C, 2 * H), f32),
            pltpu.VMEM((C, 2 * H), f32),
            pltpu.VMEM((N_ET, H), f32),
            pltpu.VMEM((C, H), f32),
            pltpu.VMEM_SHARED((N_PAD, H), f32),
            pltpu.SemaphoreType.DMA,
            pltpu.SemaphoreType.DMA,
            pltpu.SemaphoreType.DMA,
        ],
    )
    ex, dparts = pass_a(src3, dst3, et3, elr, ee_table, z8)

    inv = pl.pallas_call(
        _tc_inv_body,
        grid=(8,),
        in_specs=[pl.BlockSpec((NC, N_PAD // 8, H), lambda i: (0, i, 0))],
        out_specs=pl.BlockSpec((N_PAD // 8, H), lambda i: (i, 0)),
        out_shape=jax.ShapeDtypeStruct((N_PAD, H), f32),
    )(dparts)

    pass_b = pl.kernel(
        _sc_b_body,
        mesh=mesh,
        compiler_params=sc_params,
        out_type=[
            jax.ShapeDtypeStruct((E // 2, 2 * H), f32),
            jax.ShapeDtypeStruct((NC, N_PAD, HD), f32),
        ],
        scratch_types=[
            pltpu.VMEM((NCHUNK, C), jnp.int32),
            pltpu.VMEM((NCHUNK, C), jnp.int32),
            pltpu.VMEM((C, H), f32),
            pltpu.VMEM((C, H), f32),
            pltpu.VMEM((C, HD), f32),
            pltpu.VMEM((C, H), f32),
            pltpu.VMEM((C, H), f32),
            pltpu.VMEM((C, HD), f32),
            pltpu.VMEM((C // 2, 16), f32),
            pltpu.VMEM_SHARED((N_PAD, HD), f32),
            pltpu.SemaphoreType.DMA,
            pltpu.SemaphoreType.DMA,
            pltpu.SemaphoreType.DMA,
        ],
    )
    a_flat, rparts = pass_b(src3, dst3, ex, inv, fs)

    rst_flat = pl.pallas_call(
        _tc_add_body,
        grid=(NB,),
        in_specs=[pl.BlockSpec((NC, BR, HD), lambda i: (0, i, 0))],
        out_specs=pl.BlockSpec((BR, HD), lambda i: (i, 0)),
        out_shape=jax.ShapeDtypeStruct((N, HD), f32),
    )(rparts)

    return rst_flat.reshape(N, H, D_OUT), a_flat.reshape(E, H, 1)
